# baseline (speedup 1.0000x reference)
import functools

import jax
import jax.numpy as jnp
from jax import lax
from jax.experimental import pallas as pl
from jax.experimental.pallas import tpu as pltpu
from jax.experimental.pallas import tpu_sc as plsc

B = 16384
D = 64
NC = 2
NS = 16
NW = NC * NS
BPW = B // NW
CHUNK = 128
NCH = BPW // CHUNK


def _sc_gather_body(idx0, idx1, idx2, idx3, idx4, idx5,
                    author_t, doc_t, venue_t, affil_t,
                    out0, out1, out2, out3, out4, out5,
                    idx_v, rows_v, sem):
    wid = lax.axis_index("s") * NC + lax.axis_index("c")
    row0 = wid * NCH

    jobs = ((idx0, author_t, out0),
            (idx1, doc_t, out1),
            (idx2, doc_t, out2),
            (idx3, author_t, out3),
            (idx4, venue_t, out4),
            (idx5, affil_t, out5))

    for idx_hbm, table_hbm, out_hbm in jobs:
        pltpu.sync_copy(idx_hbm.at[pl.ds(row0, NCH)], idx_v)
        descs = []
        for j in range(NCH):
            descs.append(pltpu.async_copy(
                table_hbm.at[idx_v.at[j]],
                rows_v.at[pl.ds(j * CHUNK, CHUNK)],
                sem))
        for dsc in descs:
            dsc.wait()
        pltpu.sync_copy(rows_v, out_hbm.at[pl.ds(wid * BPW, BPW)])


@functools.cache
def _make_sc_gather():
    return pl.kernel(
        _sc_gather_body,
        mesh=plsc.VectorSubcoreMesh(core_axis_name="c", subcore_axis_name="s"),
        out_type=[jax.ShapeDtypeStruct((B, D), jnp.float32)] * 6,
        scratch_types=[
            pltpu.VMEM((NCH, CHUNK), jnp.int32),
            pltpu.VMEM((BPW, D), jnp.float32),
            pltpu.SemaphoreType.DMA,
        ],
        compiler_params=pltpu.CompilerParams(use_tc_tiling_on_sc=False),
    )


_TC_BLOCK = 1024


def _tc_bcast_body(rel_ref, o0, o1, o2, o3, o4):
    rel = rel_ref[...]
    for k, o in enumerate((o0, o1, o2, o3, o4)):
        o[...] = jnp.broadcast_to(rel[k][None, :], (_TC_BLOCK, D))


def _tc_bcast(relation_table):
    return pl.pallas_call(
        _tc_bcast_body,
        grid=(B // _TC_BLOCK,),
        in_specs=[pl.BlockSpec((5, D), lambda i: (0, 0))],
        out_specs=[pl.BlockSpec((_TC_BLOCK, D), lambda i: (i, 0))] * 5,
        out_shape=[jax.ShapeDtypeStruct((B, D), jnp.float32)] * 5,
    )(relation_table)


def kernel(user_id, wrote, cited, coauthor, venue, affiliation,
           author_table, venue_table, affiliation_table, relation_table,
           doc_embs):
    def prep(ix):
        return ix.astype(jnp.int32).reshape(B // CHUNK, CHUNK)

    outs = _make_sc_gather()(prep(user_id), prep(wrote), prep(cited), prep(coauthor),
                      prep(venue), prep(affiliation),
                      author_table, doc_embs, venue_table, affiliation_table)
    user_e, wrote_e, cited_e, coauthor_e, venue_e, affil_e = outs
    wrote_r, cited_r, coauth_r, venue_r, affil_r = _tc_bcast(relation_table)
    return (user_e, wrote_e, cited_e, coauthor_e, venue_e, affil_e,
            wrote_r, cited_r, coauth_r, venue_r, affil_r)
